# baseline (speedup 1.0000x reference)
import functools

import jax
import jax.numpy as jnp
from jax import lax
from jax.experimental import pallas as pl
from jax.experimental.pallas import tpu as pltpu
from jax.experimental.pallas import tpu_sc as plsc

B = 16
A = 4096
FD = 128
L = 16
NK = FD // L
CHUNK = 256
SCAT = 128
TOTCHUNK = A // CHUNK
NSLOT = TOTCHUNK // 2
TRASH = 8


def _mol_gather_kernel(nf_hbm, va_hbm, vf_hbm, out_hbm,
                       buf0, buf1, sc_vmem, idx_lo, idx_hi, row_buf, shared,
                       sem0, sem1):
    core = lax.axis_index("c")
    sub = lax.axis_index("s")
    b = core * (B // 2) + sub // 2
    half = sub % 2
    acc = sub // 2
    idx16 = lax.iota(jnp.int32, L)

    zero = jnp.zeros((L,), jnp.float32)
    for k in range(NK):
        row_buf[pl.ds(k * L, L)] = zero
    pltpu.sync_copy(row_buf, shared.at[sub])

    pltpu.sync_copy(va_hbm, sc_vmem.at[0, pl.ds(0, L)])
    pltpu.sync_copy(vf_hbm, sc_vmem.at[1, pl.ds(0, L)])
    va_b = sc_vmem[0, pl.ds(b, L)][0]
    vf_b = sc_vmem[1, pl.ds(b, L)][0]

    plsc.subcore_barrier()

    total_chunks = (va_b + CHUNK - 1) // CHUNK
    nc0 = (total_chunks + 1) // 2
    my_lo = jnp.where(half == 0, 0, nc0)
    my_hi = jnp.where(half == 0, nc0, total_chunks)
    end_row = jnp.minimum(my_hi * CHUNK, va_b)

    bufs = [buf0, buf1]
    sems = [sem0, sem1]

    def dma_start(i):
        g = my_lo + i

        @pl.when(g < my_hi)
        def _():
            pltpu.async_copy(
                nf_hbm.at[b, pl.ds(g * CHUNK, CHUNK), :], bufs[i % 2],
                sems[i % 2])

    dma_start(0)
    for i in range(NSLOT):
        if i + 1 < NSLOT:
            dma_start(i + 1)
        g = my_lo + i
        m = jnp.clip(end_row - g * CHUNK, 0, CHUNK)
        buf = bufs[i % 2]

        @pl.when(g < my_hi)
        def _(buf=buf, g=g, m=m):
            pltpu.make_async_copy(
                nf_hbm.at[b, pl.ds(g * CHUNK, CHUNK), :], buf,
                sems[i % 2]).wait()
            for k in range(SCAT // L):
                lane = idx16 + k * L
                idx_lo[pl.ds(k * L, L)] = jnp.where(lane < m, acc, TRASH)
            pltpu.sync_copy(buf.at[pl.ds(0, SCAT)], shared.at[idx_lo],
                            add=True)

        @pl.when(m > SCAT)
        def _(buf=buf, m=m):
            for k in range(SCAT // L):
                lane = idx16 + k * L + SCAT
                idx_hi[pl.ds(k * L, L)] = jnp.where(lane < m, acc, TRASH)
            pltpu.sync_copy(buf.at[pl.ds(SCAT, SCAT)], shared.at[idx_hi],
                            add=True)

    plsc.subcore_barrier()

    @pl.when(half == 0)
    def _():
        pltpu.sync_copy(shared.at[acc], row_buf)
        for k in range(NK):
            tot = row_buf[pl.ds(k * L, L)]
            keep = (idx16 + k * L) < vf_b
            val = jnp.maximum(jnp.where(keep, tot, jnp.float32(0.0)),
                              jnp.float32(0.0))
            row_buf[pl.ds(k * L, L)] = val
        pltpu.sync_copy(row_buf, out_hbm.at[b])


@jax.jit
def _run(node_features, valid_atoms, valid_feats):
    mesh = plsc.VectorSubcoreMesh(core_axis_name="c", subcore_axis_name="s")
    fn = functools.partial(
        pl.kernel,
        mesh=mesh,
        out_type=jax.ShapeDtypeStruct((B, FD), jnp.float32),
        scratch_types=[
            pltpu.VMEM((CHUNK, FD), jnp.float32),
            pltpu.VMEM((CHUNK, FD), jnp.float32),
            pltpu.VMEM((2, 2 * L), jnp.int32),
            pltpu.VMEM((SCAT,), jnp.int32),
            pltpu.VMEM((SCAT,), jnp.int32),
            pltpu.VMEM((FD,), jnp.float32),
            pltpu.VMEM_SHARED((2 * TRASH, FD), jnp.float32),
            pltpu.SemaphoreType.DMA,
            pltpu.SemaphoreType.DMA,
        ],
    )(_mol_gather_kernel)
    return fn(node_features, valid_atoms, valid_feats)


def kernel(node_features, data_slice):
    ds32 = data_slice.astype(jnp.int32)
    valid_atoms = ds32[:, 0]
    valid_feats = ds32[:, 1]
    return _run(node_features, valid_atoms, valid_feats)


# speedup vs baseline: 1.0342x; 1.0342x over previous
import functools

import jax
import jax.numpy as jnp
from jax import lax
from jax.experimental import pallas as pl
from jax.experimental.pallas import tpu as pltpu
from jax.experimental.pallas import tpu_sc as plsc

B = 16
A = 4096
FD = 128
L = 16
NK = FD // L
CHUNK = 256
MOLC = A // CHUNK
MPC = B // 2
NSUB = 16
MAXSLOT = MPC * MOLC // NSUB
UNROLL = 8
PARTROWS = MPC * MOLC


def _mol_gather_kernel(nf_hbm, ds_hbm, out_hbm,
                       buf0, buf1, stage, rowp, comb, row_buf, shared,
                       sem0, sem1, sem_r):
    core = lax.axis_index("c")
    sub = lax.axis_index("s")
    idx16 = lax.iota(jnp.int32, L)

    pltpu.sync_copy(ds_hbm, stage.at[pl.ds(0, 2 * B)])

    def va_of(j):
        return stage[pl.ds(2 * (MPC * core + j), L)][0]

    va_l = [va_of(j) for j in range(MPC)]
    cum = [jnp.int32(0)]
    for j in range(MPC):
        cum.append(cum[j] + (va_l[j] + CHUNK - 1) // CHUNK)
    total = cum[MPC]

    q = total // NSUB
    r = total % NSUB
    my_cnt = q + jnp.where(sub < r, 1, 0)
    my_start = sub * q + jnp.minimum(sub, r)

    def chunk_info(i):
        g = my_start + i
        lb = jnp.int32(0)
        ce = jnp.int32(0)
        va = va_l[0]
        for j in range(1, MPC):
            after = cum[j] <= g
            lb = lb + after.astype(jnp.int32)
            ce = jnp.where(after, cum[j], ce)
            va = jnp.where(after, va_l[j], va)
        jc = g - ce
        m = jnp.clip(va - jc * CHUNK, 0, CHUNK)
        return g, MPC * core + lb, jc, m

    bufs = [buf0, buf1]
    sems = [sem0, sem1]

    def dma_start(i):
        g, b, jc, m = chunk_info(i)

        @pl.when(i < my_cnt)
        def _():
            pltpu.async_copy(
                nf_hbm.at[b, pl.ds(jc * CHUNK, CHUNK), :], bufs[i % 2],
                sems[i % 2])

    dma_start(0)
    for i in range(MAXSLOT):
        if i + 1 < MAXSLOT:
            dma_start(i + 1)
        g, b, jc, m = chunk_info(i)
        buf = bufs[i % 2]

        @pl.when(i < my_cnt)
        def _(buf=buf, g=g, b=b, jc=jc, m=m, i=i):
            pltpu.make_async_copy(
                nf_hbm.at[b, pl.ds(jc * CHUNK, CHUNK), :], buf,
                sems[i % 2]).wait()
            accs = tuple(jnp.zeros((L,), jnp.float32) for _ in range(NK))

            def body(it, acc, buf=buf, m=m):
                base = it * UNROLL
                for rr in range(UNROLL):
                    j = base + rr
                    keep = j < m
                    acc = tuple(
                        acc[k] + jnp.where(keep, buf[j, pl.ds(k * L, L)],
                                           jnp.float32(0.0))
                        for k in range(NK))
                return acc

            ngroups = (m + UNROLL - 1) // UNROLL
            accs = lax.fori_loop(0, ngroups, body, accs)
            for k in range(NK):
                rowp[i, pl.ds(k * L, L)] = accs[k]
            pltpu.async_copy(rowp.at[i], shared.at[g], sem_r)

    for i in range(MAXSLOT):
        g, b, jc, m = chunk_info(i)

        @pl.when(i < my_cnt)
        def _(g=g, i=i):
            pltpu.make_async_copy(rowp.at[i], shared.at[g], sem_r).wait()
    plsc.subcore_barrier()

    @pl.when(sub < MPC)
    def _():
        b_out = MPC * core + sub
        cs = jnp.int32(0)
        n = jnp.int32(0)
        for j in range(MPC):
            here = sub == j
            cs = jnp.where(here, cum[j], cs)
            n = jnp.where(here, cum[j + 1] - cum[j], n)
        vf_b = stage[pl.ds(2 * b_out + 1, L)][0]
        pltpu.sync_copy(shared.at[pl.ds(cs, MOLC)], comb)
        accs = tuple(jnp.zeros((L,), jnp.float32) for _ in range(NK))
        for rr in range(MOLC):
            keep = rr < n
            accs = tuple(
                accs[k] + jnp.where(keep, comb[rr, pl.ds(k * L, L)],
                                    jnp.float32(0.0))
                for k in range(NK))
        for k in range(NK):
            fkeep = (idx16 + k * L) < vf_b
            row_buf[pl.ds(k * L, L)] = jnp.maximum(
                jnp.where(fkeep, accs[k], jnp.float32(0.0)), jnp.float32(0.0))
        pltpu.sync_copy(row_buf, out_hbm.at[b_out])


@jax.jit
def _run(node_features, ds_flat):
    mesh = plsc.VectorSubcoreMesh(core_axis_name="c", subcore_axis_name="s")
    fn = functools.partial(
        pl.kernel,
        mesh=mesh,
        out_type=jax.ShapeDtypeStruct((B, FD), jnp.float32),
        scratch_types=[
            pltpu.VMEM((CHUNK, FD), jnp.float32),
            pltpu.VMEM((CHUNK, FD), jnp.float32),
            pltpu.VMEM((2 * B + L,), jnp.int32),
            pltpu.VMEM((MAXSLOT, FD), jnp.float32),
            pltpu.VMEM((MOLC, FD), jnp.float32),
            pltpu.VMEM((FD,), jnp.float32),
            pltpu.VMEM_SHARED((PARTROWS + MOLC, FD), jnp.float32),
            pltpu.SemaphoreType.DMA,
            pltpu.SemaphoreType.DMA,
            pltpu.SemaphoreType.DMA,
        ],
    )(_mol_gather_kernel)
    return fn(node_features, ds_flat)


def kernel(node_features, data_slice):
    ds_flat = data_slice.astype(jnp.int32).reshape(2 * B)
    return _run(node_features, ds_flat)


# speedup vs baseline: 1.0586x; 1.0236x over previous
import functools

import jax
import jax.numpy as jnp
from jax import lax
from jax.experimental import pallas as pl
from jax.experimental.pallas import tpu as pltpu
from jax.experimental.pallas import tpu_sc as plsc

B = 16
A = 4096
FD = 128
L = 16
NK = FD // L
CHUNK = 256
MOLC = A // CHUNK
MPC = B // 2
NSUB = 16
MAXSLOT = MPC * MOLC // NSUB
UNROLL = 8
PARTROWS = MPC * MOLC


def _mol_gather_kernel(nf_hbm, ds_hbm, out_hbm,
                       buf0, buf1, buf2, stage, rowp, comb, row_buf, shared,
                       sem0, sem1, sem2, sem_r):
    core = lax.axis_index("c")
    sub = lax.axis_index("s")
    idx16 = lax.iota(jnp.int32, L)

    pltpu.async_copy(ds_hbm, stage.at[pl.ds(0, 2 * B)], sem_r)
    pltpu.make_async_copy(ds_hbm, stage.at[pl.ds(0, 2 * B)], sem_r).wait()

    def va_of(j):
        return stage[pl.ds(MPC * core + j, L)][0]

    va_l = [va_of(j) for j in range(MPC)]
    cum = [jnp.int32(0)]
    for j in range(MPC):
        cum.append(cum[j] + (va_l[j] + CHUNK - 1) // CHUNK)
    total = cum[MPC]

    q = total // NSUB
    r = total % NSUB
    my_cnt = q + jnp.where(sub < r, 1, 0)
    my_start = sub * q + jnp.minimum(sub, r)

    def chunk_info(i):
        g = my_start + i
        lb = jnp.int32(0)
        ce = jnp.int32(0)
        va = va_l[0]
        for j in range(1, MPC):
            after = cum[j] <= g
            lb = lb + after.astype(jnp.int32)
            ce = jnp.where(after, cum[j], ce)
            va = jnp.where(after, va_l[j], va)
        jc = g - ce
        m = jnp.clip(va - jc * CHUNK, 0, CHUNK)
        return g, MPC * core + lb, jc, m

    bufs = [buf0, buf1, buf2]
    sems = [sem0, sem1, sem2]
    NBUF = 3

    def dma_start(i):
        g, b, jc, m = chunk_info(i)

        @pl.when(i < my_cnt)
        def _():
            pltpu.async_copy(
                nf_hbm.at[b, pl.ds(jc * CHUNK, CHUNK), :], bufs[i % NBUF],
                sems[i % NBUF])

    dma_start(0)
    dma_start(1)
    for i in range(MAXSLOT):
        if i + 2 < MAXSLOT:
            dma_start(i + 2)
        g, b, jc, m = chunk_info(i)
        buf = bufs[i % NBUF]

        @pl.when(i < my_cnt)
        def _(buf=buf, g=g, b=b, jc=jc, m=m, i=i):
            pltpu.make_async_copy(
                nf_hbm.at[b, pl.ds(jc * CHUNK, CHUNK), :], buf,
                sems[i % NBUF]).wait()
            accs = tuple(jnp.zeros((L,), jnp.float32) for _ in range(NK))

            def body(it, acc, buf=buf, m=m):
                base = it * UNROLL
                for rr in range(UNROLL):
                    j = base + rr
                    keep = j < m
                    acc = tuple(
                        acc[k] + jnp.where(keep, buf[j, pl.ds(k * L, L)],
                                           jnp.float32(0.0))
                        for k in range(NK))
                return acc

            ngroups = (m + UNROLL - 1) // UNROLL
            accs = lax.fori_loop(0, ngroups, body, accs)
            for k in range(NK):
                rowp[i, pl.ds(k * L, L)] = accs[k]
            pltpu.async_copy(rowp.at[i], shared.at[g], sem_r)

    for i in range(MAXSLOT):
        g, b, jc, m = chunk_info(i)

        @pl.when(i < my_cnt)
        def _(g=g, i=i):
            pltpu.make_async_copy(rowp.at[i], shared.at[g], sem_r).wait()
    plsc.subcore_barrier()

    @pl.when(sub < MPC)
    def _():
        b_out = MPC * core + sub
        cs = jnp.int32(0)
        n = jnp.int32(0)
        for j in range(MPC):
            here = sub == j
            cs = jnp.where(here, cum[j], cs)
            n = jnp.where(here, cum[j + 1] - cum[j], n)
        vf_b = stage[pl.ds(B + b_out, L)][0]
        pltpu.sync_copy(shared.at[pl.ds(cs, MOLC)], comb)
        accs = tuple(jnp.zeros((L,), jnp.float32) for _ in range(NK))
        for rr in range(MOLC):
            keep = rr < n
            accs = tuple(
                accs[k] + jnp.where(keep, comb[rr, pl.ds(k * L, L)],
                                    jnp.float32(0.0))
                for k in range(NK))
        for k in range(NK):
            fkeep = (idx16 + k * L) < vf_b
            row_buf[pl.ds(k * L, L)] = jnp.maximum(
                jnp.where(fkeep, accs[k], jnp.float32(0.0)), jnp.float32(0.0))
        pltpu.sync_copy(row_buf, out_hbm.at[b_out])


@jax.jit
def _run(node_features, ds_flat):
    mesh = plsc.VectorSubcoreMesh(core_axis_name="c", subcore_axis_name="s")
    fn = functools.partial(
        pl.kernel,
        mesh=mesh,
        out_type=jax.ShapeDtypeStruct((B, FD), jnp.float32),
        scratch_types=[
            pltpu.VMEM((CHUNK, FD), jnp.float32),
            pltpu.VMEM((CHUNK, FD), jnp.float32),
            pltpu.VMEM((CHUNK, FD), jnp.float32),
            pltpu.VMEM((2 * B + L,), jnp.int32),
            pltpu.VMEM((MAXSLOT, FD), jnp.float32),
            pltpu.VMEM((MOLC, FD), jnp.float32),
            pltpu.VMEM((FD,), jnp.float32),
            pltpu.VMEM_SHARED((PARTROWS + MOLC, FD), jnp.float32),
            pltpu.SemaphoreType.DMA,
            pltpu.SemaphoreType.DMA,
            pltpu.SemaphoreType.DMA,
            pltpu.SemaphoreType.DMA,
        ],
    )(_mol_gather_kernel)
    return fn(node_features, ds_flat)


def kernel(node_features, data_slice):
    ds = data_slice.astype(jnp.int32)
    ds_flat = jnp.concatenate([ds[:, 0], ds[:, 1]])
    return _run(node_features, ds_flat)
